# baseline (speedup 1.0000x reference)
import functools

import jax
import jax.numpy as jnp
from jax import lax
from jax.experimental import pallas as pl
from jax.experimental.pallas import tpu as pltpu
from jax.experimental.pallas import tpu_sc as plsc

_L = 16
_IL = 128


def _make_sc_kernel(B, S, D, V):
    NC, NS = 2, 16
    NW = NC * NS
    RPW = B // NW
    NB = 4
    LOOKAHEAD = 2
    CH = D // _L
    REST = S - _IL
    NG = S // _L
    TAIL = S - NG * _L

    mesh = plsc.VectorSubcoreMesh(core_axis_name="c", subcore_axis_name="s")

    scratch = (
        [pltpu.VMEM((S,), jnp.int32) for _ in range(NB)]
        + [pltpu.VMEM((S, D), jnp.float32) for _ in range(NB)]
        + [pltpu.VMEM((2 * S, D), jnp.float32),
           pltpu.VMEM((2, D), jnp.float32)]
        + [pltpu.SemaphoreType.DMA for _ in range(2 * NB)]
    )

    @functools.partial(
        pl.kernel,
        out_type=jax.ShapeDtypeStruct((B, S, D), jnp.float32),
        mesh=mesh,
        scratch_types=scratch,
        compiler_params=pltpu.CompilerParams(use_tc_tiling_on_sc=False),
    )
    def sc_kernel(idx_hbm, tok_hbm, seg_hbm, pos_hbm, out_hbm, *refs):
        idxs = refs[0:NB]
        rows = refs[NB:2 * NB]
        comb = refs[2 * NB]
        seg_v = refs[2 * NB + 1]
        gsem = refs[2 * NB + 2:2 * NB + 2 + NB]
        osem = refs[2 * NB + 2 + NB:]

        wid = lax.axis_index("s") * NC + lax.axis_index("c")
        lane = lax.iota(jnp.int32, _L)

        pltpu.sync_copy(pos_hbm.at[pl.ds(0, S)], comb.at[pl.ds(0, S)])
        pltpu.sync_copy(pos_hbm.at[pl.ds(0, S)], comb.at[pl.ds(S, S)])
        pltpu.sync_copy(seg_hbm, seg_v)

        def _comb_body(j, carry):
            for ci in range(CH):
                sl = pl.ds(ci * _L, _L)
                comb[j, sl] = comb[j, sl] + seg_v[0, sl]
                comb[S + j, sl] = comb[S + j, sl] + seg_v[1, sl]
            return carry
        lax.fori_loop(0, S, _comb_body, 0)

        def _gather_parts(gb, b):
            return (
                (tok_hbm.at[idxs[b].at[pl.ds(0, _IL)]],
                 rows[b].at[pl.ds(0, _IL)]),
                (tok_hbm.at[idxs[b].at[pl.ds(_IL, REST)]],
                 rows[b].at[pl.ds(_IL, REST)]),
            )

        def _issue_gather(gb, b):
            pltpu.sync_copy(idx_hbm.at[gb], idxs[b])
            for src, dst in _gather_parts(gb, b):
                pltpu.async_copy(src, dst, gsem[b])

        def _wait_gather(gb, b):
            for src, dst in _gather_parts(gb, b):
                pltpu.make_async_copy(src, dst, gsem[b]).wait()

        def _drain_out(gb, b):
            pltpu.make_async_copy(rows[b], out_hbm.at[gb], osem[b]).wait()

        for p in range(LOOKAHEAD):
            _issue_gather(wid * RPW + p, p)

        def _add_group(b, base, lo):
            t16 = jnp.minimum(idxs[b][pl.ds(base, _L)], 1)
            src16 = t16 * S + base + lane
            for l in range(lo, _L):
                src = src16[l]
                il = base + l
                for ci in range(CH):
                    sl = pl.ds(ci * _L, _L)
                    plsc.addupdate(rows[b].at[il, sl], comb[src, sl])

        def _row_body(it, carry):
            for b in range(NB):
                c = it * NB + b
                gb = wid * RPW + c
                _wait_gather(gb, b)

                def _grp(g, c2):
                    _add_group(b, g * _L, 0)
                    return c2
                lax.fori_loop(0, NG, _grp, 0)
                if TAIL:
                    _add_group(b, S - _L, _L - TAIL)

                pltpu.async_copy(rows[b], out_hbm.at[gb], osem[b])

                nc = c + LOOKAHEAD
                nb2 = (b + LOOKAHEAD) % NB

                @pl.when(nc < RPW)
                def _ahead():
                    @pl.when(c >= LOOKAHEAD)
                    def _drain():
                        _drain_out(wid * RPW + nc, nb2)
                    _issue_gather(wid * RPW + nc, nb2)
            return carry
        lax.fori_loop(0, RPW // NB, _row_body, 0)

        for b in range(NB):
            _drain_out(wid * RPW, b)

    return sc_kernel


def kernel(inputs, token_table, segment_table, position_table):
    B, S = inputs.shape
    V, D = token_table.shape
    idx = inputs.astype(jnp.int32)
    k = _make_sc_kernel(B, S, D, V)
    return k(idx, token_table, segment_table, position_table)
